# baseline (speedup 1.0000x reference)
import functools

import jax
import jax.numpy as jnp
from jax import lax
from jax.experimental import pallas as pl
from jax.experimental.pallas import tpu as pltpu
from jax.experimental.pallas import tpu_sc as plsc

S, D, H, E, K = 2048, 768, 768, 8, 2
BLK = 256
NB = S * K // BLK + E
C = NB * BLK
NC, NS = 2, 16
NW = NC * NS
TPW = S // NW
RPW = C // NW
RG = RPW // 2


def _router_body(x_ref, wr_ref, pos_ref, w_ref, gid_ref, xb_ref):
    lT = lax.dot_general(wr_ref[...], x_ref[...], (((1,), (1,)), ((), ())),
                         preferred_element_type=jnp.float32)
    m = jnp.max(lT, axis=0, keepdims=True)
    ex = jnp.exp(lT - m)
    p = ex / jnp.sum(ex, axis=0, keepdims=True)
    erow = lax.broadcasted_iota(jnp.int32, (E, S), 0)
    m1 = jnp.max(p, axis=0, keepdims=True)
    i1 = jnp.min(jnp.where(p == m1, erow, E), axis=0, keepdims=True)
    p2 = jnp.where(erow == i1, -1.0, p)
    m2 = jnp.max(p2, axis=0, keepdims=True)
    i2 = jnp.min(jnp.where(p2 == m2, erow, E), axis=0, keepdims=True)
    s = m1 + m2

    c = (erow == i1).astype(jnp.int32) + (erow == i2).astype(jnp.int32)
    ic = c
    k = 1
    while k < S:
        ic = ic + jnp.concatenate(
            [jnp.zeros((E, k), jnp.int32), ic[:, :-k]], axis=1)
        k *= 2
    excl = ic - c
    tot = jnp.sum(c, axis=1, keepdims=True)
    nb = (tot + (BLK - 1)) // BLK
    nbc = nb
    k = 1
    while k < E:
        nbc = nbc + jnp.concatenate(
            [jnp.zeros((k, 1), jnp.int32), nbc[:-k, :]], axis=0)
        k *= 2
    off = (nbc - nb) * BLK

    rank1 = jnp.sum(jnp.where(erow == i1, excl, 0), axis=0, keepdims=True)
    off1 = jnp.sum(jnp.where(erow == i1, off, 0), axis=0, keepdims=True)
    rank2 = jnp.sum(jnp.where(erow == i2, excl, 0), axis=0, keepdims=True)
    off2 = jnp.sum(jnp.where(erow == i2, off, 0), axis=0, keepdims=True)
    pos_ref[...] = jnp.concatenate([off1 + rank1, off2 + rank2], axis=0)
    w_ref[...] = jnp.concatenate([m1 / s, m2 / s], axis=0)

    blane = lax.broadcasted_iota(jnp.int32, (E, 128), 1)
    gid = jnp.sum((blane >= nbc).astype(jnp.int32), axis=0, keepdims=True)
    gid = jnp.minimum(gid, E - 1)
    used = jnp.sum(jnp.where(erow[:, :1] == E - 1, nbc, 0), axis=0,
                   keepdims=True)
    lane = lax.broadcasted_iota(jnp.int32, (1, 128), 1)
    gid_ref[...] = jnp.where(lane == NB, used, gid)
    xb_ref[...] = x_ref[...].astype(jnp.bfloat16)


@jax.jit
def _router(x2d, Wr):
    return pl.pallas_call(
        _router_body,
        in_specs=[pl.BlockSpec((S, D), lambda: (0, 0)),
                  pl.BlockSpec((E, D), lambda: (0, 0))],
        out_specs=[pl.BlockSpec((K, S), lambda: (0, 0)),
                   pl.BlockSpec((K, S), lambda: (0, 0)),
                   pl.BlockSpec((1, 128), lambda: (0, 0)),
                   pl.BlockSpec((S, D), lambda: (0, 0))],
        out_shape=[jax.ShapeDtypeStruct((K, S), jnp.int32),
                   jax.ShapeDtypeStruct((K, S), jnp.float32),
                   jax.ShapeDtypeStruct((1, 128), jnp.int32),
                   jax.ShapeDtypeStruct((S, D), jnp.bfloat16)],
    )(x2d, Wr)


def _invperm_body(pos_hbm, w_hbm, tok_hbm, ws_hbm, idx_v, w_v, t_v, sem):
    wid = lax.axis_index("s") * NC + lax.axis_index("c")
    pltpu.sync_copy(pos_hbm.at[wid], idx_v)
    pltpu.sync_copy(w_hbm.at[wid], w_v)
    base = wid * TPW
    for q in range(TPW // 16):
        t_v[pl.ds(q * 16, 16)] = lax.iota(jnp.int32, 16) + (base + q * 16)
    c1 = pltpu.async_copy(t_v, tok_hbm.at[idx_v.at[0]], sem)
    c2 = pltpu.async_copy(t_v, tok_hbm.at[idx_v.at[1]], sem)
    c3 = pltpu.async_copy(w_v.at[0], ws_hbm.at[idx_v.at[0]], sem)
    c4 = pltpu.async_copy(w_v.at[1], ws_hbm.at[idx_v.at[1]], sem)
    c1.wait(); c2.wait(); c3.wait(); c4.wait()


@jax.jit
def _invperm(pos_t, w_t):
    return pl.kernel(
        _invperm_body,
        mesh=plsc.VectorSubcoreMesh(core_axis_name="c", subcore_axis_name="s"),
        out_type=[jax.ShapeDtypeStruct((C,), jnp.int32),
                  jax.ShapeDtypeStruct((C,), jnp.float32)],
        scratch_types=[pltpu.VMEM((K, TPW), jnp.int32),
                       pltpu.VMEM((K, TPW), jnp.float32),
                       pltpu.VMEM((TPW,), jnp.int32),
                       pltpu.SemaphoreType.DMA],
    )(pos_t, w_t)


def _gatherx_body(xb_hbm, tok_hbm, xs_hbm, tki_v, rows_v, sem):
    wid = lax.axis_index("s") * NC + lax.axis_index("c")
    base = wid * RPW
    pltpu.sync_copy(tok_hbm.at[pl.ds(base, RG)], tki_v.at[0])
    pltpu.sync_copy(tok_hbm.at[pl.ds(base + RG, RG)], tki_v.at[1])
    for q in range(2):
        for r in range(RG // 16):
            sl = pl.ds(r * 16, 16)
            tki_v[q, sl] = jnp.clip(tki_v[q, sl], 0, S - 1)
    g1 = pltpu.async_copy(xb_hbm.at[tki_v.at[0]], rows_v.at[pl.ds(0, RG)], sem)
    g2 = pltpu.async_copy(xb_hbm.at[tki_v.at[1]], rows_v.at[pl.ds(RG, RG)], sem)
    g1.wait(); g2.wait()
    pltpu.sync_copy(rows_v, xs_hbm.at[pl.ds(base, RPW)])


@jax.jit
def _gatherx(xb, tok):
    return pl.kernel(
        _gatherx_body,
        mesh=plsc.VectorSubcoreMesh(core_axis_name="c", subcore_axis_name="s"),
        out_type=jax.ShapeDtypeStruct((C, D // 2), jnp.int32),
        scratch_types=[pltpu.VMEM((2, RG), jnp.int32),
                       pltpu.VMEM((RPW, D // 2), jnp.int32),
                       pltpu.SemaphoreType.DMA],
    )(xb, tok)


def _ffn_body(g_ref, xs_ref, ws_ref, w1_ref, b1_ref, w2_ref, b2_ref, y_ref):
    b = pl.program_id(0)

    @pl.when(b < g_ref[NB])
    def _():
        xsf = xs_ref[...].astype(jnp.float32)
        h = lax.dot_general(xsf, w1_ref[0], (((1,), (1,)), ((), ())),
                            preferred_element_type=jnp.float32)
        h = jnp.maximum(h + b1_ref[0], 0.0)
        y = lax.dot_general(h, w2_ref[0], (((1,), (1,)), ((), ())),
                            preferred_element_type=jnp.float32)
        y_ref[...] = (y + b2_ref[0]) * ws_ref[...]


@jax.jit
def _ffn(gid, xs, ws, W1, b1, W2, b2):
    gs = pltpu.PrefetchScalarGridSpec(
        num_scalar_prefetch=1,
        grid=(NB,),
        in_specs=[
            pl.BlockSpec((BLK, D), lambda b, g: (b, 0)),
            pl.BlockSpec((BLK, 1), lambda b, g: (b, 0)),
            pl.BlockSpec((1, H, D), lambda b, g: (g[b], 0, 0)),
            pl.BlockSpec((1, 1, H), lambda b, g: (g[b], 0, 0)),
            pl.BlockSpec((1, D, H), lambda b, g: (g[b], 0, 0)),
            pl.BlockSpec((1, 1, D), lambda b, g: (g[b], 0, 0)),
        ],
        out_specs=pl.BlockSpec((BLK, D), lambda b, g: (b, 0)),
    )
    return pl.pallas_call(
        _ffn_body,
        grid_spec=gs,
        out_shape=jax.ShapeDtypeStruct((C, D), jnp.float32),
    )(gid, xs, ws.reshape(C, 1), W1, b1.reshape(E, 1, H), W2,
      b2.reshape(E, 1, D))


def _combine_body(y_hbm, pos_hbm, o_hbm, idx_v, ya_v, yb_v, sem):
    wid = lax.axis_index("s") * NC + lax.axis_index("c")
    pltpu.sync_copy(pos_hbm.at[wid], idx_v)
    g1 = pltpu.async_copy(y_hbm.at[idx_v.at[0]], ya_v, sem)
    g2 = pltpu.async_copy(y_hbm.at[idx_v.at[1]], yb_v, sem)
    g1.wait(); g2.wait()

    def row(j, carry):
        for cc in range(D // 16):
            sl = pl.ds(cc * 16, 16)
            ya_v[j, sl] = ya_v[j, sl] + yb_v[j, sl]
        return carry

    lax.fori_loop(0, TPW, row, 0)
    pltpu.sync_copy(ya_v, o_hbm.at[pl.ds(wid * TPW, TPW)])


@jax.jit
def _combine(y, pos_t):
    return pl.kernel(
        _combine_body,
        mesh=plsc.VectorSubcoreMesh(core_axis_name="c", subcore_axis_name="s"),
        out_type=jax.ShapeDtypeStruct((S, D), jnp.float32),
        scratch_types=[pltpu.VMEM((K, TPW), jnp.int32),
                       pltpu.VMEM((TPW, D), jnp.float32),
                       pltpu.VMEM((TPW, D), jnp.float32),
                       pltpu.SemaphoreType.DMA],
    )(y, pos_t)


def kernel(x, Wr, W1, b1, W2, b2):
    Bs, Ss, Ds = x.shape
    x2d = x.reshape(Ss, Ds)
    pos, w, gidU, xb = _router(x2d, Wr)
    pos_t = pos.reshape(K, NW, TPW).transpose(1, 0, 2)
    w_t = w.reshape(K, NW, TPW).transpose(1, 0, 2)
    tok, ws = _invperm(pos_t, w_t)
    xb32 = lax.bitcast_convert_type(
        lax.bitcast_convert_type(xb, jnp.uint16).reshape(Ss, Ds // 2, 2),
        jnp.int32)
    xs32 = _gatherx(xb32, tok)
    xs = lax.bitcast_convert_type(
        lax.bitcast_convert_type(xs32, jnp.uint16).reshape(C, Ds),
        jnp.bfloat16)
    y = _ffn(gidU.reshape(128), xs, ws, W1, b1, W2, b2)
    out = _combine(y, pos_t)
    return (out.reshape(Bs, Ss, Ds), jnp.float32(0.0))


# speedup vs baseline: 3.1559x; 3.1559x over previous
import functools

import jax
import jax.numpy as jnp
from jax import lax
from jax.experimental import pallas as pl
from jax.experimental.pallas import tpu as pltpu

S, D, H, E, K = 2048, 768, 768, 8, 2
TBLK = 256
NBT = S // TBLK


def _moe_body(x_ref, wr_ref, w1_ref, b1_ref, w2_ref, b2_ref, out_ref, wdense):
    e = pl.program_id(0)
    tb = pl.program_id(1)

    @pl.when((e == 0) & (tb == 0))
    def _router():
        xx = x_ref[...]
        logits = lax.dot_general(xx, wr_ref[...], (((1,), (1,)), ((), ())),
                                 preferred_element_type=jnp.float32)
        m = jnp.max(logits, axis=1, keepdims=True)
        ex = jnp.exp(logits - m)
        p = ex / jnp.sum(ex, axis=1, keepdims=True)
        lane = lax.broadcasted_iota(jnp.int32, (S, E), 1)
        m1 = jnp.max(p, axis=1, keepdims=True)
        i1 = jnp.min(jnp.where(p == m1, lane, E), axis=1, keepdims=True)
        p2 = jnp.where(lane == i1, -1.0, p)
        m2 = jnp.max(p2, axis=1, keepdims=True)
        i2 = jnp.min(jnp.where(p2 == m2, lane, E), axis=1, keepdims=True)
        s = m1 + m2
        wdense[...] = jnp.where(lane == i1, m1 / s,
                                jnp.where(lane == i2, m2 / s, 0.0))

    rows = pl.ds(tb * TBLK, TBLK)
    lane = lax.broadcasted_iota(jnp.int32, (TBLK, E), 1)
    w_e = jnp.sum(jnp.where(lane == e, wdense[rows, :], 0.0), axis=1,
                  keepdims=True)
    xt = x_ref[rows, :]
    h = lax.dot_general(xt, w1_ref[0], (((1,), (1,)), ((), ())),
                        preferred_element_type=jnp.float32)
    h = jnp.maximum(h + b1_ref[0], 0.0)
    y = lax.dot_general(h, w2_ref[0], (((1,), (1,)), ((), ())),
                        preferred_element_type=jnp.float32)
    contrib = w_e * (y + b2_ref[0])

    @pl.when(e == 0)
    def _init():
        out_ref[rows, :] = contrib

    @pl.when(e > 0)
    def _acc():
        out_ref[rows, :] = out_ref[rows, :] + contrib


@jax.jit
def _moe(x2d, Wr, W1, b1, W2, b2):
    return pl.pallas_call(
        _moe_body,
        grid=(E, NBT),
        in_specs=[
            pl.BlockSpec((S, D), lambda e, tb: (0, 0)),
            pl.BlockSpec((E, D), lambda e, tb: (0, 0)),
            pl.BlockSpec((1, H, D), lambda e, tb: (e, 0, 0)),
            pl.BlockSpec((1, 1, H), lambda e, tb: (e, 0, 0)),
            pl.BlockSpec((1, D, H), lambda e, tb: (e, 0, 0)),
            pl.BlockSpec((1, 1, D), lambda e, tb: (e, 0, 0)),
        ],
        out_specs=pl.BlockSpec((S, D), lambda e, tb: (0, 0)),
        out_shape=jax.ShapeDtypeStruct((S, D), jnp.float32),
        scratch_shapes=[pltpu.VMEM((S, E), jnp.float32)],
    )(x2d, Wr, W1, b1.reshape(E, 1, H), W2, b2.reshape(E, 1, D))


def kernel(x, Wr, W1, b1, W2, b2):
    Bs, Ss, Ds = x.shape
    out = _moe(x.reshape(Ss, Ds), Wr, W1, b1, W2, b2)
    return (out.reshape(Bs, Ss, Ds), jnp.float32(0.0))


# speedup vs baseline: 4.8050x; 1.5226x over previous
import functools

import jax
import jax.numpy as jnp
from jax import lax
from jax.experimental import pallas as pl
from jax.experimental.pallas import tpu as pltpu

S, D, H, E, K = 2048, 768, 768, 8, 2


def _moe_body(x_ref, wr_ref, w1_ref, b1_ref, w2_ref, b2_ref, out_ref, wdense):
    e = pl.program_id(0)

    @pl.when(e == 0)
    def _router():
        xx = x_ref[...]
        logits = lax.dot_general(xx, wr_ref[...], (((1,), (1,)), ((), ())),
                                 preferred_element_type=jnp.float32)
        m = jnp.max(logits, axis=1, keepdims=True)
        ex = jnp.exp(logits - m)
        p = ex / jnp.sum(ex, axis=1, keepdims=True)
        lane = lax.broadcasted_iota(jnp.int32, (S, E), 1)
        m1 = jnp.max(p, axis=1, keepdims=True)
        i1 = jnp.min(jnp.where(p == m1, lane, E), axis=1, keepdims=True)
        p2 = jnp.where(lane == i1, -1.0, p)
        m2 = jnp.max(p2, axis=1, keepdims=True)
        i2 = jnp.min(jnp.where(p2 == m2, lane, E), axis=1, keepdims=True)
        s = m1 + m2
        wdense[...] = jnp.where(lane == i1, m1 / s,
                                jnp.where(lane == i2, m2 / s, 0.0))

    lane = lax.broadcasted_iota(jnp.int32, (S, E), 1)
    w_e = jnp.sum(jnp.where(lane == e, wdense[...], 0.0), axis=1,
                  keepdims=True)
    h = lax.dot_general(x_ref[...], w1_ref[0], (((1,), (1,)), ((), ())),
                        preferred_element_type=jnp.float32)
    h = jnp.maximum(h + b1_ref[0], 0.0)
    y = lax.dot_general(h, w2_ref[0], (((1,), (1,)), ((), ())),
                        preferred_element_type=jnp.float32)
    y = y + b2_ref[0]

    @pl.when(e == 0)
    def _init():
        out_ref[...] = w_e * y

    @pl.when(e > 0)
    def _acc():
        out_ref[...] = out_ref[...] + w_e * y


@jax.jit
def _moe(x2d, Wr, W1, b1, W2, b2):
    return pl.pallas_call(
        _moe_body,
        grid=(E,),
        in_specs=[
            pl.BlockSpec((S, D), lambda e: (0, 0)),
            pl.BlockSpec((E, D), lambda e: (0, 0)),
            pl.BlockSpec((1, H, D), lambda e: (e, 0, 0)),
            pl.BlockSpec((1, 1, H), lambda e: (e, 0, 0)),
            pl.BlockSpec((1, D, H), lambda e: (e, 0, 0)),
            pl.BlockSpec((1, 1, D), lambda e: (e, 0, 0)),
        ],
        out_specs=pl.BlockSpec((S, D), lambda e: (0, 0)),
        out_shape=jax.ShapeDtypeStruct((S, D), jnp.float32),
        scratch_shapes=[pltpu.VMEM((S, E), jnp.float32)],
    )(x2d, Wr, W1, b1.reshape(E, 1, H), W2, b2.reshape(E, 1, D))


def kernel(x, Wr, W1, b1, W2, b2):
    Bs, Ss, Ds = x.shape
    out = _moe(x.reshape(Ss, Ds), Wr, W1, b1, W2, b2)
    return (out.reshape(Bs, Ss, Ds), jnp.float32(0.0))
